# baseline (speedup 1.0000x reference)
import functools

import jax
import jax.numpy as jnp
from jax import lax
from jax.experimental import pallas as pl
from jax.experimental.pallas import tpu as pltpu
from jax.experimental.pallas import tpu_sc as plsc

_B = 1048576
_ALPHA = 0.25
_INT_MIN = -2147483648

_ZROWS = 2 * _B // 128
_TROWS = _B // 128

_NT = 16
_PER = _B // _NT
_NV = _PER // 16
_ZCH_ROWS = 64
_ECH = _ZCH_ROWS * 64
_NCH = _PER // _ECH
_CIT = _ECH // 16


def _sc_body(z_hbm, tg_hbm, out_hbm,
             keys_v, zbuf_v, tbuf_v, hist_v, counts_v, merge_v, ivec_v,
             sh_hist):
    sid = lax.axis_index("s")
    lane = lax.iota(jnp.int32, 16)
    ones = jnp.ones((16,), jnp.int32)

    def zh(j, _):
        hist_v[pl.ds(j * 16, 16)] = jnp.zeros((16,), jnp.int32)
        return 0
    lax.fori_loop(0, 1024, zh, 0)

    npos = jnp.zeros((16,), jnp.int32)
    for c in range(_NCH):
        pltpu.sync_copy(
            z_hbm.at[pl.ds(sid * 1024 + c * _ZCH_ROWS, _ZCH_ROWS)], zbuf_v)
        pltpu.sync_copy(
            tg_hbm.at[pl.ds(sid * (_PER // 128) + c * (_ZCH_ROWS // 2),
                            _ZCH_ROWS // 2)], tbuf_v)

        def cbody(i4, np_acc):
            for s in range(4):
                i = i4 * 4 + s
                m = lax.shift_right_logical(i, 3)
                j = (i & 7) * 16
                l0 = zbuf_v[2 * m, pl.ds(j, 16)]
                l1 = zbuf_v[2 * m + 1, pl.ds(j, 16)]
                t = tbuf_v[m, pl.ds(j, 16)]
                d = l1 - l0
                bits = lax.bitcast_convert_type(d, jnp.int32)
                key = bits ^ (lax.shift_right_arithmetic(bits, 31)
                              & jnp.int32(0x7FFFFFFF))
                pos = t == 1
                key = jnp.where(pos, jnp.int32(_INT_MIN), key)
                keys_v[pl.ds(c * _ECH + i * 16, 16)] = key
                dig = (lax.shift_right_arithmetic(key, 24) & 255) ^ 128
                plsc.addupdate_scatter(hist_v,
                                       [s * 4096 + dig * 16 + lane], ones)
                np_acc = np_acc + jnp.where(pos, 1, 0)
            return np_acc
        npos = lax.fori_loop(0, _CIT // 4, cbody, npos)

    k_rem = jnp.zeros((16,), jnp.int32)
    hi = jnp.zeros((16,), jnp.int32)
    num_pos = jnp.zeros((16,), jnp.int32)
    k_full = jnp.zeros((16,), jnp.int32)

    for level, shift in enumerate((24, 16, 8, 0)):
        if level > 0:
            def zh2(j, _):
                hist_v[pl.ds(j * 16, 16)] = jnp.zeros((16,), jnp.int32)
                return 0
            lax.fori_loop(0, 1024, zh2, 0)

            hi_l = hi

            def sbody(i4, _):
                for s in range(4):
                    i = i4 * 4 + s
                    kv = keys_v[pl.ds(i * 16, 16)]
                    dig = lax.shift_right_arithmetic(kv, shift) & 255
                    pm = lax.shift_right_arithmetic(kv, shift + 8) == hi_l
                    plsc.addupdate_scatter(hist_v,
                                           [s * 4096 + dig * 16 + lane],
                                           ones, mask=pm)
                return 0
            lax.fori_loop(0, _NV // 4, sbody, 0)

        def rbody(g, _):
            bidx = (g * 16 + lane) * 16
            acc = jnp.zeros((16,), jnp.int32)
            for u in range(4):
                for j in range(16):
                    acc = acc + plsc.load_gather(hist_v, [u * 4096 + bidx + j])
            counts_v[pl.ds(g * 16, 16)] = acc
            return 0
        lax.fori_loop(0, 16, rbody, 0)
        counts_v[pl.ds(256, 16)] = npos

        pltpu.sync_copy(counts_v, sh_hist.at[sid])
        plsc.subcore_barrier()
        pltpu.sync_copy(sh_hist, merge_v)
        plsc.subcore_barrier()

        if level == 0:
            npv = jnp.zeros((16,), jnp.int32)
            for tt in range(16):
                npv = npv + merge_v[tt, pl.ds(256, 16)]
            num_pos = jnp.full((16,), jnp.sum(npv), jnp.int32)
            k_full = lax.shift_right_logical(num_pos, 1)
            k_rem = jnp.maximum(k_full, 1)

        carry = jnp.zeros((16,), jnp.int32)
        dv = jnp.full((16,), 256, jnp.int32)
        cv = jnp.full((16,), -1, jnp.int32)
        for g in range(15, -1, -1):
            gc = jnp.zeros((16,), jnp.int32)
            for tt in range(16):
                gc = gc + merge_v[tt, pl.ds(g * 16, 16)]
            cum = plsc.cumsum(gc)
            tot = jnp.sum(gc)
            gg = carry + (tot - cum)
            sel = gg < k_rem
            dvec = g * 16 + lane
            dv = jnp.minimum(dv, jnp.where(sel, dvec, 256))
            cv = jnp.maximum(cv, jnp.where(sel, gg, -1))
            carry = carry + tot
        dd = jnp.min(dv)
        cd = jnp.max(cv)
        k_rem = k_rem - cd
        if level == 0:
            hi = jnp.full((16,), dd, jnp.int32) - 128
        else:
            hi = hi * 256 + dd

    @pl.when(sid == 0)
    def _():
        for r, vec in ((0, hi), (1, k_rem), (2, num_pos), (3, k_full)):
            for g in range(8):
                ivec_v[pl.ds(g * 16, 16)] = vec
            pltpu.sync_copy(ivec_v, out_hbm.at[r])


@functools.cache
def _get_sc_call():
  mesh = plsc.VectorSubcoreMesh(core_axis_name="c", subcore_axis_name="s",
                                num_cores=1, num_subcores=16)
  return functools.partial(
    pl.kernel,
    out_type=jax.ShapeDtypeStruct((8, 128), jnp.int32),
    mesh=mesh,
    compiler_params=pltpu.CompilerParams(needs_layout_passes=False),
    scratch_types=[
        pltpu.VMEM((_PER,), jnp.int32),
        pltpu.VMEM((_ZCH_ROWS, 128), jnp.float32),
        pltpu.VMEM((_ZCH_ROWS // 2, 128), jnp.int32),
        pltpu.VMEM((16384,), jnp.int32),
        pltpu.VMEM((272,), jnp.int32),
        pltpu.VMEM((16, 272), jnp.int32),
        pltpu.VMEM((128,), jnp.int32),
        pltpu.MemorySpace.HBM((16, 272), jnp.int32),
    ],
  )(_sc_body)


_BRT = 512
_GRID = _TROWS // _BRT


def _tc_body(z_ref, tg_ref, sel_ref, out_ref):
    i = pl.program_id(0)
    x = z_ref[...]
    l0 = x[:, 0, :]
    l1 = x[:, 1, :]
    t = tg_ref[...]
    d = l1 - l0

    pos = t == 1
    s = jnp.where(pos, -d, d)
    ce = jnp.maximum(s, 0.0) + jnp.log1p(jnp.exp(-jnp.abs(s)))
    pt = jnp.exp(-ce)
    focal = _ALPHA * (1.0 - pt) * (1.0 - pt) * ce

    bits = lax.bitcast_convert_type(d, jnp.int32)
    key = bits ^ (lax.shift_right_arithmetic(bits, 31) & jnp.int32(0x7FFFFFFF))
    key = jnp.where(pos, jnp.int32(_INT_MIN), key)

    kstar = sel_ref[0, 0]
    posf = pos.astype(jnp.float32)
    gt = (key > kstar).astype(jnp.float32)
    eqf = jnp.where(key == kstar, focal, 0.0)

    lanev = lax.broadcasted_iota(jnp.int32, (1, 128), 1)
    sums = (jnp.where(lanev == 0, jnp.sum(posf), 0.0)
            + jnp.where(lanev == 1, jnp.sum(focal * posf), 0.0)
            + jnp.where(lanev == 2, jnp.sum(focal), 0.0)
            + jnp.where(lanev == 3, jnp.sum(focal * gt), 0.0))
    bmax = jnp.where(lanev == 4, jnp.max(eqf), 0.0)

    @pl.when(i == 0)
    def _():
        out_ref[...] = jnp.zeros_like(out_ref)

    cur = out_ref[...]
    new = jnp.where(lanev == 4, jnp.maximum(cur, bmax), cur + sums)
    out_ref[...] = new

    @pl.when(i == _GRID - 1)
    def _():
        st = out_ref[...]
        npos_f = st[0, 0]
        sum_pos = st[0, 1]
        total = st[0, 2]
        s_gt = st[0, 3]
        f_at = st[0, 4]
        krem_f = sel_ref[1, 0].astype(jnp.float32)
        k_f = sel_ref[3, 0].astype(jnp.float32)
        hard = (sum_pos + s_gt + krem_f * f_at) / (npos_f + k_f)
        meanv = total * (1.0 / _B)
        num_neg_f = jnp.float32(_B) - npos_f
        use_h = (k_f > 0.0) & (num_neg_f > k_f)
        fin = jnp.where(use_h, hard, meanv)
        out_ref[...] = jnp.where(lanev == 7, fin, st)


_tc_call = pl.pallas_call(
    _tc_body,
    grid=(_GRID,),
    in_specs=[
        pl.BlockSpec((_BRT, 2, 128), lambda i: (i, 0, 0)),
        pl.BlockSpec((_BRT, 128), lambda i: (i, 0)),
        pl.BlockSpec((8, 128), lambda i: (0, 0)),
    ],
    out_specs=pl.BlockSpec((1, 128), lambda i: (0, 0)),
    out_shape=jax.ShapeDtypeStruct((1, 128), jnp.float32),
)


def kernel(logits, targets):
    z2d = logits.reshape(_TROWS, 128, 2).swapaxes(1, 2).reshape(_ZROWS, 128)
    tg2d = targets.reshape(_TROWS, 128)

    z3d = z2d.reshape(_TROWS, 2, 128)
    sel = _get_sc_call()(z2d, tg2d)
    out = _tc_call(z3d, tg2d, sel)
    return out[0, 7]


# speedup vs baseline: 1.2100x; 1.2100x over previous
import functools

import jax
import jax.numpy as jnp
from jax import lax
from jax.experimental import pallas as pl
from jax.experimental.pallas import tpu as pltpu
from jax.experimental.pallas import tpu_sc as plsc

_B = 1048576
_ALPHA = 0.25
_INT_MIN = -2147483648

_ZROWS = 2 * _B // 128
_TROWS = _B // 128

_NT = 16
_PER = _B // _NT
_NV = _PER // 16
_ZCH_ROWS = 64
_ECH = _ZCH_ROWS * 64
_NCH = _PER // _ECH
_CIT = _ECH // 16


def _sc_body(z_hbm, tg_hbm, out_hbm,
             keys_v, zbuf_v, tbuf_v, hist_v, counts_v, merge_v, ivec_v,
             sh_hist, zsem, tsem):
    sid = lax.axis_index("s")
    lane = lax.iota(jnp.int32, 16)
    ones = jnp.ones((16,), jnp.int32)

    def zh(j, _):
        hist_v[pl.ds(j * 16, 16)] = jnp.zeros((16,), jnp.int32)
        return 0
    lax.fori_loop(0, 256, zh, 0)

    def _start(c):
        b = c & 1
        pltpu.make_async_copy(
            z_hbm.at[pl.ds(sid * 1024 + c * _ZCH_ROWS, _ZCH_ROWS)],
            zbuf_v.at[b], zsem.at[b]).start()
        pltpu.make_async_copy(
            tg_hbm.at[pl.ds(sid * (_PER // 128) + c * (_ZCH_ROWS // 2),
                            _ZCH_ROWS // 2)],
            tbuf_v.at[b], tsem.at[b]).start()

    def _wait(c):
        b = c & 1
        pltpu.make_async_copy(
            z_hbm.at[pl.ds(sid * 1024 + c * _ZCH_ROWS, _ZCH_ROWS)],
            zbuf_v.at[b], zsem.at[b]).wait()
        pltpu.make_async_copy(
            tg_hbm.at[pl.ds(sid * (_PER // 128) + c * (_ZCH_ROWS // 2),
                            _ZCH_ROWS // 2)],
            tbuf_v.at[b], tsem.at[b]).wait()

    npos = jnp.zeros((16,), jnp.int32)
    _start(0)
    for c in range(_NCH):
        if c + 1 < _NCH:
            _start(c + 1)
        _wait(c)
        b = c & 1

        def cbody(i4, np_acc):
            for s in range(4):
                i = i4 * 4 + s
                m = lax.shift_right_logical(i, 3)
                j = (i & 7) * 16
                l0 = zbuf_v[b, 2 * m, pl.ds(j, 16)]
                l1 = zbuf_v[b, 2 * m + 1, pl.ds(j, 16)]
                t = tbuf_v[b, m, pl.ds(j, 16)]
                d = l1 - l0
                bits = lax.bitcast_convert_type(d, jnp.int32)
                key = bits ^ (lax.shift_right_arithmetic(bits, 31)
                              & jnp.int32(0x7FFFFFFF))
                pos = t == 1
                key = jnp.where(pos, jnp.int32(_INT_MIN), key)
                keys_v[pl.ds(c * _ECH + i * 16, 16)] = key
                dig = (lax.shift_right_arithmetic(key, 24) & 255) ^ 128
                plsc.addupdate_scatter(hist_v, [dig * 16 + lane], ones)
                np_acc = np_acc + jnp.where(pos, 1, 0)
            return np_acc
        npos = lax.fori_loop(0, _CIT // 4, cbody, npos)

    k_rem = jnp.zeros((16,), jnp.int32)
    hi = jnp.zeros((16,), jnp.int32)
    num_pos = jnp.zeros((16,), jnp.int32)
    k_full = jnp.zeros((16,), jnp.int32)

    for level, shift in enumerate((24, 16, 8, 0)):
        if level > 0:
            def zh2(j, _):
                hist_v[pl.ds(j * 16, 16)] = jnp.zeros((16,), jnp.int32)
                return 0
            lax.fori_loop(0, 256, zh2, 0)

            hi_l = hi

            def sbody(i4, _):
                for s in range(4):
                    i = i4 * 4 + s
                    kv = keys_v[pl.ds(i * 16, 16)]
                    dig = lax.shift_right_arithmetic(kv, shift) & 255
                    pm = lax.shift_right_arithmetic(kv, shift + 8) == hi_l
                    plsc.addupdate_scatter(hist_v, [dig * 16 + lane], ones,
                                           mask=pm)
                return 0
            lax.fori_loop(0, _NV // 4, sbody, 0)

        def rbody(g, _):
            bidx = (g * 16 + lane) * 16
            acc = jnp.zeros((16,), jnp.int32)
            for j in range(16):
                acc = acc + plsc.load_gather(hist_v, [bidx + j])
            counts_v[pl.ds(g * 16, 16)] = acc
            return 0
        lax.fori_loop(0, 16, rbody, 0)
        counts_v[pl.ds(256, 16)] = npos

        pltpu.sync_copy(counts_v, sh_hist.at[sid])
        plsc.subcore_barrier()
        pltpu.sync_copy(sh_hist, merge_v)
        plsc.subcore_barrier()

        if level == 0:
            npv = jnp.zeros((16,), jnp.int32)
            for tt in range(16):
                npv = npv + merge_v[tt, pl.ds(256, 16)]
            num_pos = jnp.full((16,), jnp.sum(npv), jnp.int32)
            k_full = lax.shift_right_logical(num_pos, 1)
            k_rem = jnp.maximum(k_full, 1)

        carry = jnp.zeros((16,), jnp.int32)
        dv = jnp.full((16,), 256, jnp.int32)
        cv = jnp.full((16,), -1, jnp.int32)
        for g in range(15, -1, -1):
            gc = jnp.zeros((16,), jnp.int32)
            for tt in range(16):
                gc = gc + merge_v[tt, pl.ds(g * 16, 16)]
            cum = plsc.cumsum(gc)
            tot = jnp.sum(gc)
            gg = carry + (tot - cum)
            sel = gg < k_rem
            dvec = g * 16 + lane
            dv = jnp.minimum(dv, jnp.where(sel, dvec, 256))
            cv = jnp.maximum(cv, jnp.where(sel, gg, -1))
            carry = carry + tot
        dd = jnp.min(dv)
        cd = jnp.max(cv)
        k_rem = k_rem - cd
        if level == 0:
            hi = jnp.full((16,), dd, jnp.int32) - 128
        else:
            hi = hi * 256 + dd

    @pl.when(sid == 0)
    def _():
        for r, vec in ((0, hi), (1, k_rem), (2, num_pos), (3, k_full)):
            for g in range(8):
                ivec_v[pl.ds(g * 16, 16)] = vec
            pltpu.sync_copy(ivec_v, out_hbm.at[r])


@functools.cache
def _get_sc_call():
  mesh = plsc.VectorSubcoreMesh(core_axis_name="c", subcore_axis_name="s",
                                num_cores=1, num_subcores=16)
  return functools.partial(
    pl.kernel,
    out_type=jax.ShapeDtypeStruct((8, 128), jnp.int32),
    mesh=mesh,
    compiler_params=pltpu.CompilerParams(needs_layout_passes=False),
    scratch_types=[
        pltpu.VMEM((_PER,), jnp.int32),
        pltpu.VMEM((2, _ZCH_ROWS, 128), jnp.float32),
        pltpu.VMEM((2, _ZCH_ROWS // 2, 128), jnp.int32),
        pltpu.VMEM((4096,), jnp.int32),
        pltpu.VMEM((272,), jnp.int32),
        pltpu.VMEM((16, 272), jnp.int32),
        pltpu.VMEM((128,), jnp.int32),
        pltpu.MemorySpace.HBM((16, 272), jnp.int32),
        pltpu.SemaphoreType.DMA((2,)),
        pltpu.SemaphoreType.DMA((2,)),
    ],
  )(_sc_body)


_BRT = 512
_GRID = _TROWS // _BRT


def _tc_body(z_ref, tg_ref, sel_ref, out_ref):
    i = pl.program_id(0)
    x = z_ref[...]
    l0 = x[:, 0, :]
    l1 = x[:, 1, :]
    t = tg_ref[...]
    d = l1 - l0

    pos = t == 1
    s = jnp.where(pos, -d, d)
    ce = jnp.maximum(s, 0.0) + jnp.log1p(jnp.exp(-jnp.abs(s)))
    pt = jnp.exp(-ce)
    focal = _ALPHA * (1.0 - pt) * (1.0 - pt) * ce

    bits = lax.bitcast_convert_type(d, jnp.int32)
    key = bits ^ (lax.shift_right_arithmetic(bits, 31) & jnp.int32(0x7FFFFFFF))
    key = jnp.where(pos, jnp.int32(_INT_MIN), key)

    kstar = sel_ref[0, 0]
    posf = pos.astype(jnp.float32)
    gt = (key > kstar).astype(jnp.float32)
    eqf = jnp.where(key == kstar, focal, 0.0)

    lanev = lax.broadcasted_iota(jnp.int32, (1, 128), 1)
    sums = (jnp.where(lanev == 0, jnp.sum(posf), 0.0)
            + jnp.where(lanev == 1, jnp.sum(focal * posf), 0.0)
            + jnp.where(lanev == 2, jnp.sum(focal), 0.0)
            + jnp.where(lanev == 3, jnp.sum(focal * gt), 0.0))
    bmax = jnp.where(lanev == 4, jnp.max(eqf), 0.0)

    @pl.when(i == 0)
    def _():
        out_ref[...] = jnp.zeros_like(out_ref)

    cur = out_ref[...]
    new = jnp.where(lanev == 4, jnp.maximum(cur, bmax), cur + sums)
    out_ref[...] = new

    @pl.when(i == _GRID - 1)
    def _():
        st = out_ref[...]
        npos_f = st[0, 0]
        sum_pos = st[0, 1]
        total = st[0, 2]
        s_gt = st[0, 3]
        f_at = st[0, 4]
        krem_f = sel_ref[1, 0].astype(jnp.float32)
        k_f = sel_ref[3, 0].astype(jnp.float32)
        hard = (sum_pos + s_gt + krem_f * f_at) / (npos_f + k_f)
        meanv = total * (1.0 / _B)
        num_neg_f = jnp.float32(_B) - npos_f
        use_h = (k_f > 0.0) & (num_neg_f > k_f)
        fin = jnp.where(use_h, hard, meanv)
        out_ref[...] = jnp.where(lanev == 7, fin, st)


_tc_call = pl.pallas_call(
    _tc_body,
    grid=(_GRID,),
    in_specs=[
        pl.BlockSpec((_BRT, 2, 128), lambda i: (i, 0, 0)),
        pl.BlockSpec((_BRT, 128), lambda i: (i, 0)),
        pl.BlockSpec((8, 128), lambda i: (0, 0)),
    ],
    out_specs=pl.BlockSpec((1, 128), lambda i: (0, 0)),
    out_shape=jax.ShapeDtypeStruct((1, 128), jnp.float32),
)


def kernel(logits, targets):
    z2d = logits.reshape(_TROWS, 128, 2).swapaxes(1, 2).reshape(_ZROWS, 128)
    tg2d = targets.reshape(_TROWS, 128)

    z3d = z2d.reshape(_TROWS, 2, 128)
    sel = _get_sc_call()(z2d, tg2d)
    out = _tc_call(z3d, tg2d, sel)
    return out[0, 7]


# speedup vs baseline: 1.2339x; 1.0198x over previous
import functools

import jax
import jax.numpy as jnp
from jax import lax
from jax.experimental import pallas as pl
from jax.experimental.pallas import tpu as pltpu
from jax.experimental.pallas import tpu_sc as plsc

_B = 1048576
_ALPHA = 0.25
_INT_MIN = -2147483648

_ZROWS = 2 * _B // 128
_TROWS = _B // 128

_NT = 16
_PER = _B // _NT
_NV = _PER // 16
_ZCH_ROWS = 64
_ECH = _ZCH_ROWS * 64
_NCH = _PER // _ECH
_CIT = _ECH // 16


def _sc_body(z_hbm, tg_hbm, out_hbm,
             keys_v, zbuf_v, tbuf_v, hist_v, counts_v, merge_v, ivec_v,
             sh_hist, zsem, tsem):
    sid = lax.axis_index("s")
    lane = lax.iota(jnp.int32, 16)
    ones = jnp.ones((16,), jnp.int32)

    def zh(j, _):
        hist_v[pl.ds(j * 16, 16)] = jnp.zeros((16,), jnp.int32)
        return 0
    lax.fori_loop(0, 256, zh, 0)

    def _start(c):
        b = c & 1
        pltpu.make_async_copy(
            z_hbm.at[pl.ds(sid * 1024 + c * _ZCH_ROWS, _ZCH_ROWS)],
            zbuf_v.at[b], zsem.at[b]).start()
        pltpu.make_async_copy(
            tg_hbm.at[pl.ds(sid * (_PER // 128) + c * (_ZCH_ROWS // 2),
                            _ZCH_ROWS // 2)],
            tbuf_v.at[b], tsem.at[b]).start()

    def _wait(c):
        b = c & 1
        pltpu.make_async_copy(
            z_hbm.at[pl.ds(sid * 1024 + c * _ZCH_ROWS, _ZCH_ROWS)],
            zbuf_v.at[b], zsem.at[b]).wait()
        pltpu.make_async_copy(
            tg_hbm.at[pl.ds(sid * (_PER // 128) + c * (_ZCH_ROWS // 2),
                            _ZCH_ROWS // 2)],
            tbuf_v.at[b], tsem.at[b]).wait()

    npos = jnp.zeros((16,), jnp.int32)
    _start(0)
    for c in range(_NCH):
        if c + 1 < _NCH:
            _start(c + 1)
        _wait(c)
        b = c & 1

        def cbody(i4, np_acc):
            for s in range(8):
                i = i4 * 8 + s
                m = lax.shift_right_logical(i, 3)
                j = (i & 7) * 16
                l0 = zbuf_v[b, 2 * m, pl.ds(j, 16)]
                l1 = zbuf_v[b, 2 * m + 1, pl.ds(j, 16)]
                t = tbuf_v[b, m, pl.ds(j, 16)]
                d = l1 - l0
                bits = lax.bitcast_convert_type(d, jnp.int32)
                key = bits ^ (lax.shift_right_arithmetic(bits, 31)
                              & jnp.int32(0x7FFFFFFF))
                pos = t == 1
                key = jnp.where(pos, jnp.int32(_INT_MIN), key)
                keys_v[pl.ds(c * _ECH + i * 16, 16)] = key
                dig = (lax.shift_right_arithmetic(key, 24) & 255) ^ 128
                plsc.addupdate_scatter(hist_v, [dig * 16 + lane], ones)
                np_acc = np_acc + jnp.where(pos, 1, 0)
            return np_acc
        npos = lax.fori_loop(0, _CIT // 8, cbody, npos)

    k_rem = jnp.zeros((16,), jnp.int32)
    hi = jnp.zeros((16,), jnp.int32)
    num_pos = jnp.zeros((16,), jnp.int32)
    k_full = jnp.zeros((16,), jnp.int32)

    for level, shift in enumerate((24, 16, 8, 0)):
        if level > 0:
            def zh2(j, _):
                hist_v[pl.ds(j * 16, 16)] = jnp.zeros((16,), jnp.int32)
                return 0
            lax.fori_loop(0, 256, zh2, 0)

            hi_l = hi

            def sbody(i4, _):
                for s in range(8):
                    i = i4 * 8 + s
                    kv = keys_v[pl.ds(i * 16, 16)]
                    dig = lax.shift_right_arithmetic(kv, shift) & 255
                    pm = lax.shift_right_arithmetic(kv, shift + 8) == hi_l
                    plsc.addupdate_scatter(hist_v, [dig * 16 + lane], ones,
                                           mask=pm)
                return 0
            lax.fori_loop(0, _NV // 8, sbody, 0)

        def rbody(g, _):
            bidx = (g * 16 + lane) * 16
            acc = jnp.zeros((16,), jnp.int32)
            for j in range(16):
                acc = acc + plsc.load_gather(hist_v, [bidx + j])
            counts_v[pl.ds(g * 16, 16)] = acc
            return 0
        lax.fori_loop(0, 16, rbody, 0)
        counts_v[pl.ds(256, 16)] = npos

        pltpu.sync_copy(counts_v, sh_hist.at[sid])
        plsc.subcore_barrier()
        pltpu.sync_copy(sh_hist, merge_v)
        plsc.subcore_barrier()

        if level == 0:
            npv = jnp.zeros((16,), jnp.int32)
            for tt in range(16):
                npv = npv + merge_v[tt, pl.ds(256, 16)]
            num_pos = jnp.full((16,), jnp.sum(npv), jnp.int32)
            k_full = lax.shift_right_logical(num_pos, 1)
            k_rem = jnp.maximum(k_full, 1)

        carry = jnp.zeros((16,), jnp.int32)
        dv = jnp.full((16,), 256, jnp.int32)
        cv = jnp.full((16,), -1, jnp.int32)
        for g in range(15, -1, -1):
            gc = jnp.zeros((16,), jnp.int32)
            for tt in range(16):
                gc = gc + merge_v[tt, pl.ds(g * 16, 16)]
            cum = plsc.cumsum(gc)
            tot = jnp.sum(gc)
            gg = carry + (tot - cum)
            sel = gg < k_rem
            dvec = g * 16 + lane
            dv = jnp.minimum(dv, jnp.where(sel, dvec, 256))
            cv = jnp.maximum(cv, jnp.where(sel, gg, -1))
            carry = carry + tot
        dd = jnp.min(dv)
        cd = jnp.max(cv)
        k_rem = k_rem - cd
        if level == 0:
            hi = jnp.full((16,), dd, jnp.int32) - 128
        else:
            hi = hi * 256 + dd

    @pl.when(sid == 0)
    def _():
        for r, vec in ((0, hi), (1, k_rem), (2, num_pos), (3, k_full)):
            for g in range(8):
                ivec_v[pl.ds(g * 16, 16)] = vec
            pltpu.sync_copy(ivec_v, out_hbm.at[r])


@functools.cache
def _get_sc_call():
  mesh = plsc.VectorSubcoreMesh(core_axis_name="c", subcore_axis_name="s",
                                num_cores=1, num_subcores=16)
  return functools.partial(
    pl.kernel,
    out_type=jax.ShapeDtypeStruct((8, 128), jnp.int32),
    mesh=mesh,
    compiler_params=pltpu.CompilerParams(needs_layout_passes=False),
    scratch_types=[
        pltpu.VMEM((_PER,), jnp.int32),
        pltpu.VMEM((2, _ZCH_ROWS, 128), jnp.float32),
        pltpu.VMEM((2, _ZCH_ROWS // 2, 128), jnp.int32),
        pltpu.VMEM((4096,), jnp.int32),
        pltpu.VMEM((272,), jnp.int32),
        pltpu.VMEM((16, 272), jnp.int32),
        pltpu.VMEM((128,), jnp.int32),
        pltpu.MemorySpace.HBM((16, 272), jnp.int32),
        pltpu.SemaphoreType.DMA((2,)),
        pltpu.SemaphoreType.DMA((2,)),
    ],
  )(_sc_body)


_BRT = 512
_GRID = _TROWS // _BRT


def _tc_body(z_ref, tg_ref, sel_ref, out_ref):
    i = pl.program_id(0)
    x = z_ref[...]
    l0 = x[:, 0, :]
    l1 = x[:, 1, :]
    t = tg_ref[...]
    d = l1 - l0

    pos = t == 1
    s = jnp.where(pos, -d, d)
    ce = jnp.maximum(s, 0.0) + jnp.log1p(jnp.exp(-jnp.abs(s)))
    pt = jnp.exp(-ce)
    focal = _ALPHA * (1.0 - pt) * (1.0 - pt) * ce

    bits = lax.bitcast_convert_type(d, jnp.int32)
    key = bits ^ (lax.shift_right_arithmetic(bits, 31) & jnp.int32(0x7FFFFFFF))
    key = jnp.where(pos, jnp.int32(_INT_MIN), key)

    kstar = sel_ref[0, 0]
    posf = pos.astype(jnp.float32)
    gt = (key > kstar).astype(jnp.float32)
    eqf = jnp.where(key == kstar, focal, 0.0)

    lanev = lax.broadcasted_iota(jnp.int32, (1, 128), 1)
    sums = (jnp.where(lanev == 0, jnp.sum(posf), 0.0)
            + jnp.where(lanev == 1, jnp.sum(focal * posf), 0.0)
            + jnp.where(lanev == 2, jnp.sum(focal), 0.0)
            + jnp.where(lanev == 3, jnp.sum(focal * gt), 0.0))
    bmax = jnp.where(lanev == 4, jnp.max(eqf), 0.0)

    @pl.when(i == 0)
    def _():
        out_ref[...] = jnp.zeros_like(out_ref)

    cur = out_ref[...]
    new = jnp.where(lanev == 4, jnp.maximum(cur, bmax), cur + sums)
    out_ref[...] = new

    @pl.when(i == _GRID - 1)
    def _():
        st = out_ref[...]
        npos_f = st[0, 0]
        sum_pos = st[0, 1]
        total = st[0, 2]
        s_gt = st[0, 3]
        f_at = st[0, 4]
        krem_f = sel_ref[1, 0].astype(jnp.float32)
        k_f = sel_ref[3, 0].astype(jnp.float32)
        hard = (sum_pos + s_gt + krem_f * f_at) / (npos_f + k_f)
        meanv = total * (1.0 / _B)
        num_neg_f = jnp.float32(_B) - npos_f
        use_h = (k_f > 0.0) & (num_neg_f > k_f)
        fin = jnp.where(use_h, hard, meanv)
        out_ref[...] = jnp.where(lanev == 7, fin, st)


_tc_call = pl.pallas_call(
    _tc_body,
    grid=(_GRID,),
    in_specs=[
        pl.BlockSpec((_BRT, 2, 128), lambda i: (i, 0, 0)),
        pl.BlockSpec((_BRT, 128), lambda i: (i, 0)),
        pl.BlockSpec((8, 128), lambda i: (0, 0)),
    ],
    out_specs=pl.BlockSpec((1, 128), lambda i: (0, 0)),
    out_shape=jax.ShapeDtypeStruct((1, 128), jnp.float32),
)


def kernel(logits, targets):
    z2d = logits.reshape(_TROWS, 128, 2).swapaxes(1, 2).reshape(_ZROWS, 128)
    tg2d = targets.reshape(_TROWS, 128)

    z3d = z2d.reshape(_TROWS, 2, 128)
    sel = _get_sc_call()(z2d, tg2d)
    out = _tc_call(z3d, tg2d, sel)
    return out[0, 7]
